# baseline (speedup 1.0000x reference)
import functools

import jax
import jax.numpy as jnp
from jax import lax
from jax.experimental import pallas as pl
from jax.experimental.pallas import tpu as pltpu
from jax.experimental.pallas import tpu_sc as plsc

B = 16384
N_SPARSE = 26
N_CONT = 13
VOCAB = 100000
EMB_DIM = 16
EMB_COLS = N_SPARSE * EMB_DIM

NUM_CORES = 2
NUM_SUBCORES = 16
NW = NUM_CORES * NUM_SUBCORES
COLS_PER_W = EMB_COLS // NW
CH = 4096
NCH = B // CH


def _sc_col_gather(tabT, tail, idxT):
    mesh = plsc.VectorSubcoreMesh(core_axis_name="c", subcore_axis_name="s")

    @functools.partial(
        pl.kernel,
        out_type=jax.ShapeDtypeStruct((EMB_COLS, B), jnp.float32),
        mesh=mesh,
        scratch_types=[
            pltpu.VMEM((33536,), jnp.float32),
            pltpu.VMEM((33536,), jnp.float32),
            pltpu.VMEM((B,), jnp.int32),
            pltpu.VMEM((B,), jnp.float32),
            pltpu.VMEM((B,), jnp.float32),
            pltpu.SemaphoreType.DMA,
            pltpu.SemaphoreType.DMA,
        ],
        compiler_params=pltpu.CompilerParams(
            use_tc_tiling_on_sc=True, needs_layout_passes=False
        ),
    )
    def col_gather(tabT_hbm, tail_hbm, idxT_hbm, outT_hbm, vocab_a, vocab_b, idxf_v, out_a, out_b, vsem, wsem):
        vocabs = (vocab_a, vocab_b)
        outs = (out_a, out_b)
        wid = lax.axis_index("s") * NUM_CORES + lax.axis_index("c")
        TH = (0, 33408, 66816)
        SZ = (33408, 33408, 33152)
        HI = (33408, 66816, VOCAB)
        NSTEP = COLS_PER_W * 3

        def start_dma(s):
            j, t = divmod(s, 3)
            r = wid * COLS_PER_W + j
            ds = [
                pltpu.async_copy(
                    tabT_hbm.at[r, pl.ds(TH[t], SZ[t])],
                    vocabs[s % 2].at[pl.ds(0, SZ[t])],
                    vsem,
                )
            ]
            if t == 2:
                ds.append(
                    pltpu.async_copy(
                        tail_hbm.at[r],
                        vocabs[s % 2].at[pl.ds(SZ[t], 128)],
                        vsem,
                    )
                )
            return ds

        dmas = [start_dma(0), None]
        writes = [None, None]
        for s in range(NSTEP):
            j, t = divmod(s, 3)
            r = wid * COLS_PER_W + j
            f = r // EMB_DIM
            if t == 0:
                if j == 0:
                    pltpu.sync_copy(idxT_hbm.at[f], idxf_v)
                else:
                    f_prev = (wid * COLS_PER_W + j - 1) // EMB_DIM

                    @pl.when(f != f_prev)
                    def _():
                        pltpu.sync_copy(idxT_hbm.at[f], idxf_v)

                if writes[j % 2] is not None:
                    writes[j % 2].wait()
                    writes[j % 2] = None
            if s + 1 < NSTEP:
                dmas[(s + 1) % 2] = start_dma(s + 1)
            for d in dmas[s % 2]:
                d.wait()

            lo, hi = TH[t], HI[t]

            @plsc.parallel_loop(0, B // 16, unroll=8)
            def body(i, s=s, t=t, j=j, lo=lo, hi=hi):
                off = i * 16
                iv = idxf_v[pl.ds(off, 16)]
                m = (iv >= lo) & (iv < hi)
                ivc = jnp.clip(iv - lo, 0, hi - lo - 1)
                g = plsc.load_gather(vocabs[s % 2], [ivc])
                val = jnp.where(m, g, 0.0)
                if t == 0:
                    outs[j % 2][pl.ds(off, 16)] = val
                else:
                    plsc.addupdate(outs[j % 2].at[pl.ds(off, 16)], val)

            if t == 2:
                writes[j % 2] = pltpu.async_copy(
                    outs[j % 2], outT_hbm.at[r], wsem
                )
        for w in writes:
            if w is not None:
                w.wait()

    return col_gather(tabT, tail, idxT)


def _dense_body(
    xt_ref, ct_ref, w1a_ref, w1b_ref, b1_ref, w2_ref, b2_ref, w3_ref,
    fmwa_ref, fmwb_ref, fmva_ref, fmvb_ref, bias_ref, o_ref,
):
    embT = xt_ref[...]
    contT = ct_ref[...]
    bf = jnp.bfloat16
    embTb = embT.astype(bf)
    contTb = contT.astype(bf)

    def dott(w, x):
        return lax.dot_general(
            w.astype(bf), x, (((0,), (0,)), ((), ())),
            preferred_element_type=jnp.float32,
        )

    h1T = jnp.maximum(
        dott(w1a_ref[...], embTb) + dott(w1b_ref[...], contTb) + b1_ref[...].T, 0.0
    )
    h2T = jnp.maximum(dott(w2_ref[...], h1T.astype(bf)) + b2_ref[...].T, 0.0)
    deepT = jnp.sum(h2T * w3_ref[...], axis=0, keepdims=True)

    linT = (
        jnp.sum(embT * fmwa_ref[...], axis=0, keepdims=True)
        + jnp.sum(contT * fmwb_ref[...], axis=0, keepdims=True)
    )

    fmva = fmva_ref[...]
    fmvb = fmvb_ref[...]
    xvT = dott(fmva, embTb) + dott(fmvb, contTb)
    x2v2T = dott(fmva * fmva, (embT * embT).astype(bf)) + dott(
        fmvb * fmvb, (contT * contT).astype(bf)
    )
    interT = 0.5 * jnp.sum(xvT * xvT - x2v2T, axis=0, keepdims=True)

    o_ref[...] = linT + interT + deepT + bias_ref[...]


def kernel(deep_sparse, deep_cont, emb_tables, fm_w, fm_b, fm_v, W1, b1, W2, b2, W3, b3):
    tabT = jnp.transpose(emb_tables, (0, 2, 1)).reshape(EMB_COLS, VOCAB)
    idxT = deep_sparse.T.astype(jnp.int32)

    tail = jnp.pad(tabT[:, 99968:], ((0, 0), (0, 96)))
    xT = _sc_col_gather(tabT, tail, idxT)

    contT = jnp.pad(deep_cont.astype(jnp.float32).T, ((0, 16 - N_CONT), (0, 0)))

    w1a = W1[:EMB_COLS]
    w1b = jnp.pad(W1[EMB_COLS:], ((0, 16 - N_CONT), (0, 0)))
    fmwa = fm_w[:EMB_COLS]
    fmwb = jnp.pad(fm_w[EMB_COLS:], ((0, 16 - N_CONT), (0, 0)))
    fmva = fm_v[:EMB_COLS]
    fmvb = jnp.pad(fm_v[EMB_COLS:], ((0, 16 - N_CONT), (0, 0)))
    b1r = b1[None, :]
    b2r = b2[None, :]
    bias = (fm_b + b3).reshape(1, 1)

    BN = 2048
    full = lambda shape: pl.BlockSpec(shape, lambda i: (0, 0))
    outT = pl.pallas_call(
        _dense_body,
        grid=(B // BN,),
        in_specs=[
            pl.BlockSpec((EMB_COLS, BN), lambda i: (0, i)),
            pl.BlockSpec((16, BN), lambda i: (0, i)),
            full((EMB_COLS, 256)),
            full((16, 256)),
            full((1, 256)),
            full((256, 128)),
            full((1, 128)),
            full((128, 1)),
            full((EMB_COLS, 1)),
            full((16, 1)),
            full((EMB_COLS, 16)),
            full((16, 16)),
            full((1, 1)),
        ],
        out_specs=pl.BlockSpec((1, BN), lambda i: (0, i)),
        out_shape=jax.ShapeDtypeStruct((1, B), jnp.float32),
    )(
        xT, contT, w1a, w1b, b1r, W2, b2r, W3,
        fmwa, fmwb, fmva, fmvb, bias,
    )
    return outT.reshape(B, 1)


# speedup vs baseline: 1.0412x; 1.0412x over previous
import functools

import jax
import jax.numpy as jnp
from jax import lax
from jax.experimental import pallas as pl
from jax.experimental.pallas import tpu as pltpu
from jax.experimental.pallas import tpu_sc as plsc

B = 16384
N_SPARSE = 26
N_CONT = 13
VOCAB = 100000
EMB_DIM = 16
EMB_COLS = N_SPARSE * EMB_DIM

NUM_CORES = 2
NUM_SUBCORES = 16
NW = NUM_CORES * NUM_SUBCORES
COLS_PER_W = EMB_COLS // NW
CH = 4096
NCH = B // CH


def _sc_col_gather(tabT, tail, idxT):
    mesh = plsc.VectorSubcoreMesh(core_axis_name="c", subcore_axis_name="s")

    @functools.partial(
        pl.kernel,
        out_type=jax.ShapeDtypeStruct((EMB_COLS, B), jnp.float32),
        mesh=mesh,
        scratch_types=[
            pltpu.VMEM((VOCAB,), jnp.float32),
            pltpu.VMEM((1,), jnp.float32),
            pltpu.VMEM((B,), jnp.int32),
            pltpu.VMEM((2, CH), jnp.float32),
            pltpu.VMEM((1,), jnp.float32),
            pltpu.SemaphoreType.DMA,
            pltpu.SemaphoreType.DMA,
        ],
        compiler_params=pltpu.CompilerParams(
            use_tc_tiling_on_sc=True, needs_layout_passes=False
        ),
    )
    def col_gather(tabT_hbm, tail_hbm, idxT_hbm, outT_hbm, vocab_a, vocab_b, idxf_v, out_a, out_b, vsem, wsem):
        del vocab_b, out_b, tail_hbm
        wid = lax.axis_index("s") * NUM_CORES + lax.axis_index("c")
        UNROLL = 8
        writes = [None, None]
        for j in range(COLS_PER_W):
            r = wid * COLS_PER_W + j
            f = r // EMB_DIM
            if j == 0:
                pltpu.sync_copy(idxT_hbm.at[f], idxf_v)
            else:
                f_prev = (wid * COLS_PER_W + j - 1) // EMB_DIM

                @pl.when(f != f_prev)
                def _():
                    pltpu.sync_copy(idxT_hbm.at[f], idxf_v)

            pltpu.sync_copy(tabT_hbm.at[r], vocab_a)
            for c in range(NCH):
                buf = c % 2
                if writes[buf] is not None:
                    writes[buf].wait()
                    writes[buf] = None

                @plsc.parallel_loop(0, CH // 16, unroll=16)
                def body(i, buf=buf, c=c):
                    off = i * 16
                    iv = idxf_v[pl.ds(c * CH + off, 16)]
                    out_a[buf, pl.ds(off, 16)] = plsc.load_gather(
                        vocab_a, [iv]
                    )
                writes[buf] = pltpu.async_copy(
                    out_a.at[buf], outT_hbm.at[r, pl.ds(c * CH, CH)], wsem
                )
        for w in writes:
            if w is not None:
                w.wait()

    return col_gather(tabT, tail, idxT)


def _dense_body(
    xt_ref, ct_ref, w1a_ref, w1b_ref, b1_ref, w2_ref, b2_ref, w3_ref,
    fmwa_ref, fmwb_ref, fmva_ref, fmvb_ref, bias_ref, o_ref,
):
    embT = xt_ref[...]
    contT = ct_ref[...]
    bf = jnp.bfloat16
    embTb = embT.astype(bf)
    contTb = contT.astype(bf)

    def dott(w, x):
        return lax.dot_general(
            w.astype(bf), x, (((0,), (0,)), ((), ())),
            preferred_element_type=jnp.float32,
        )

    h1T = jnp.maximum(
        dott(w1a_ref[...], embTb) + dott(w1b_ref[...], contTb) + b1_ref[...].T, 0.0
    )
    h2T = jnp.maximum(dott(w2_ref[...], h1T.astype(bf)) + b2_ref[...].T, 0.0)
    deepT = jnp.sum(h2T * w3_ref[...], axis=0, keepdims=True)

    linT = (
        jnp.sum(embT * fmwa_ref[...], axis=0, keepdims=True)
        + jnp.sum(contT * fmwb_ref[...], axis=0, keepdims=True)
    )

    fmva = fmva_ref[...]
    fmvb = fmvb_ref[...]
    xvT = dott(fmva, embTb) + dott(fmvb, contTb)
    x2v2T = dott(fmva * fmva, (embT * embT).astype(bf)) + dott(
        fmvb * fmvb, (contT * contT).astype(bf)
    )
    interT = 0.5 * jnp.sum(xvT * xvT - x2v2T, axis=0, keepdims=True)

    o_ref[...] = linT + interT + deepT + bias_ref[...]


def kernel(deep_sparse, deep_cont, emb_tables, fm_w, fm_b, fm_v, W1, b1, W2, b2, W3, b3):
    tabT = jnp.transpose(emb_tables, (0, 2, 1)).reshape(EMB_COLS, VOCAB)
    idxT = deep_sparse.T.astype(jnp.int32)

    tail = jnp.pad(tabT[:, 99968:], ((0, 0), (0, 96)))
    xT = _sc_col_gather(tabT, tail, idxT)

    contT = jnp.pad(deep_cont.astype(jnp.float32).T, ((0, 16 - N_CONT), (0, 0)))

    w1a = W1[:EMB_COLS]
    w1b = jnp.pad(W1[EMB_COLS:], ((0, 16 - N_CONT), (0, 0)))
    fmwa = fm_w[:EMB_COLS]
    fmwb = jnp.pad(fm_w[EMB_COLS:], ((0, 16 - N_CONT), (0, 0)))
    fmva = fm_v[:EMB_COLS]
    fmvb = jnp.pad(fm_v[EMB_COLS:], ((0, 16 - N_CONT), (0, 0)))
    b1r = b1[None, :]
    b2r = b2[None, :]
    bias = (fm_b + b3).reshape(1, 1)

    BN = 2048
    full = lambda shape: pl.BlockSpec(shape, lambda i: (0, 0))
    outT = pl.pallas_call(
        _dense_body,
        grid=(B // BN,),
        in_specs=[
            pl.BlockSpec((EMB_COLS, BN), lambda i: (0, i)),
            pl.BlockSpec((16, BN), lambda i: (0, i)),
            full((EMB_COLS, 256)),
            full((16, 256)),
            full((1, 256)),
            full((256, 128)),
            full((1, 128)),
            full((128, 1)),
            full((EMB_COLS, 1)),
            full((16, 1)),
            full((EMB_COLS, 16)),
            full((16, 16)),
            full((1, 1)),
        ],
        out_specs=pl.BlockSpec((1, BN), lambda i: (0, i)),
        out_shape=jax.ShapeDtypeStruct((1, B), jnp.float32),
    )(
        xT, contT, w1a, w1b, b1r, W2, b2r, W3,
        fmwa, fmwb, fmva, fmvb, bias,
    )
    return outT.reshape(B, 1)


# speedup vs baseline: 1.0646x; 1.0225x over previous
import functools

import jax
import jax.numpy as jnp
from jax import lax
from jax.experimental import pallas as pl
from jax.experimental.pallas import tpu as pltpu
from jax.experimental.pallas import tpu_sc as plsc

B = 16384
N_SPARSE = 26
N_CONT = 13
VOCAB = 100000
EMB_DIM = 16
EMB_COLS = N_SPARSE * EMB_DIM

NUM_CORES = 2
NUM_SUBCORES = 16
NW = NUM_CORES * NUM_SUBCORES
COLS_PER_W = EMB_COLS // NW
CH = 4096
NCH = B // CH


def _sc_col_gather(tabT, tail, idxT):
    mesh = plsc.VectorSubcoreMesh(core_axis_name="c", subcore_axis_name="s")

    @functools.partial(
        pl.kernel,
        out_type=jax.ShapeDtypeStruct((EMB_COLS, B), jnp.float32),
        mesh=mesh,
        scratch_types=[
            pltpu.VMEM((VOCAB,), jnp.float32),
            pltpu.VMEM((1,), jnp.float32),
            pltpu.VMEM((B,), jnp.int32),
            pltpu.VMEM((2, CH), jnp.float32),
            pltpu.VMEM((1,), jnp.float32),
            pltpu.SemaphoreType.DMA,
            pltpu.SemaphoreType.DMA,
        ],
        compiler_params=pltpu.CompilerParams(
            use_tc_tiling_on_sc=True, needs_layout_passes=False
        ),
    )
    def col_gather(tabT_hbm, tail_hbm, idxT_hbm, outT_hbm, vocab_a, vocab_b, idxf_v, out_a, out_b, vsem, wsem):
        del vocab_b, out_b, tail_hbm
        wid = lax.axis_index("s") * NUM_CORES + lax.axis_index("c")
        UNROLL = 8
        writes = [None, None]
        for j in range(COLS_PER_W):
            r = wid * COLS_PER_W + j
            f = r // EMB_DIM
            if j == 0:
                pltpu.sync_copy(idxT_hbm.at[f], idxf_v)
            else:
                f_prev = (wid * COLS_PER_W + j - 1) // EMB_DIM

                @pl.when(f != f_prev)
                def _():
                    pltpu.sync_copy(idxT_hbm.at[f], idxf_v)

            pltpu.sync_copy(tabT_hbm.at[r], vocab_a)
            for c in range(NCH):
                buf = c % 2
                if writes[buf] is not None:
                    writes[buf].wait()
                    writes[buf] = None

                @plsc.parallel_loop(0, CH // 16, unroll=UNROLL)
                def body(i, buf=buf, c=c):
                    off = i * 16
                    iv = idxf_v[pl.ds(c * CH + off, 16)]
                    out_a[buf, pl.ds(off, 16)] = plsc.load_gather(
                        vocab_a, [iv]
                    )
                writes[buf] = pltpu.async_copy(
                    out_a.at[buf], outT_hbm.at[r, pl.ds(c * CH, CH)], wsem
                )
        for w in writes:
            if w is not None:
                w.wait()

    return col_gather(tabT, tail, idxT)


def _dense_body(
    xt_ref, ct_ref, w1a_ref, w1b_ref, b1_ref, w2_ref, b2_ref, w3_ref,
    fmwa_ref, fmwb_ref, fmva_ref, fmvb_ref, bias_ref, o_ref,
):
    embT = xt_ref[...]
    contT = ct_ref[...]
    bf = jnp.bfloat16
    embTb = embT.astype(bf)
    contTb = contT.astype(bf)

    def dott(w, x):
        return lax.dot_general(
            w.astype(bf), x, (((0,), (0,)), ((), ())),
            preferred_element_type=jnp.float32,
        )

    h1T = jnp.maximum(
        dott(w1a_ref[...], embTb) + dott(w1b_ref[...], contTb) + b1_ref[...].T, 0.0
    )
    h2T = jnp.maximum(dott(w2_ref[...], h1T.astype(bf)) + b2_ref[...].T, 0.0)
    deepT = jnp.sum(h2T * w3_ref[...], axis=0, keepdims=True)

    linT = (
        jnp.sum(embT * fmwa_ref[...], axis=0, keepdims=True)
        + jnp.sum(contT * fmwb_ref[...], axis=0, keepdims=True)
    )

    fmva = fmva_ref[...]
    fmvb = fmvb_ref[...]
    xvT = dott(fmva, embTb) + dott(fmvb, contTb)
    x2v2T = dott(fmva * fmva, (embT * embT).astype(bf)) + dott(
        fmvb * fmvb, (contT * contT).astype(bf)
    )
    interT = 0.5 * jnp.sum(xvT * xvT - x2v2T, axis=0, keepdims=True)

    o_ref[...] = linT + interT + deepT + bias_ref[...]


def kernel(deep_sparse, deep_cont, emb_tables, fm_w, fm_b, fm_v, W1, b1, W2, b2, W3, b3):
    tabT = jnp.transpose(emb_tables, (0, 2, 1)).reshape(EMB_COLS, VOCAB)
    idxT = deep_sparse.T.astype(jnp.int32)

    tail = jnp.pad(tabT[:, 99968:], ((0, 0), (0, 96)))
    xT = _sc_col_gather(tabT, tail, idxT)

    contT = jnp.pad(deep_cont.astype(jnp.float32).T, ((0, 16 - N_CONT), (0, 0)))

    w1a = W1[:EMB_COLS]
    w1b = jnp.pad(W1[EMB_COLS:], ((0, 16 - N_CONT), (0, 0)))
    fmwa = fm_w[:EMB_COLS]
    fmwb = jnp.pad(fm_w[EMB_COLS:], ((0, 16 - N_CONT), (0, 0)))
    fmva = fm_v[:EMB_COLS]
    fmvb = jnp.pad(fm_v[EMB_COLS:], ((0, 16 - N_CONT), (0, 0)))
    b1r = b1[None, :]
    b2r = b2[None, :]
    bias = (fm_b + b3).reshape(1, 1)

    BN = 2048
    full = lambda shape: pl.BlockSpec(shape, lambda i: (0, 0))
    outT = pl.pallas_call(
        _dense_body,
        grid=(B // BN,),
        in_specs=[
            pl.BlockSpec((EMB_COLS, BN), lambda i: (0, i)),
            pl.BlockSpec((16, BN), lambda i: (0, i)),
            full((EMB_COLS, 256)),
            full((16, 256)),
            full((1, 256)),
            full((256, 128)),
            full((1, 128)),
            full((128, 1)),
            full((EMB_COLS, 1)),
            full((16, 1)),
            full((EMB_COLS, 16)),
            full((16, 16)),
            full((1, 1)),
        ],
        out_specs=pl.BlockSpec((1, BN), lambda i: (0, i)),
        out_shape=jax.ShapeDtypeStruct((1, B), jnp.float32),
    )(
        xT, contT, w1a, w1b, b1r, W2, b2r, W3,
        fmwa, fmwb, fmva, fmvb, bias,
    )
    return outT.reshape(B, 1)
